# baseline (speedup 1.0000x reference)
import functools

import jax
import jax.numpy as jnp
from jax import lax
from jax.experimental import pallas as pl
from jax.experimental.pallas import tpu as pltpu
from jax.experimental.pallas import tpu_sc as plsc

NUM_NODES = 100000
MEM_DIM = 128
MSG_DIM = 100
B = 4096
PERIOD = 4
C = MSG_DIM + MEM_DIM

NC = 2
NS = 16
NW = NC * NS
ROWS_PER_W = B // NW
L = 16

_ROWS = 1000
_NSTEP = NUM_NODES // _ROWS
_D = 4
_BLK = 512
_NBLK = B // _BLK


def _worker_id():
  return lax.axis_index("s") * NC + lax.axis_index("c")


@functools.cache
def _get_sc_kernels():
  mesh = plsc.VectorSubcoreMesh(
      core_axis_name="c", subcore_axis_name="s", num_cores=NC)

  @functools.partial(
      pl.kernel,
      out_type=(
          jax.ShapeDtypeStruct((B, MEM_DIM), jnp.float32),
          jax.ShapeDtypeStruct((B,), jnp.int32),
      ),
      mesh=mesh,
      compiler_params=pltpu.CompilerParams(needs_layout_passes=False),
      scratch_types=[
          pltpu.VMEM((ROWS_PER_W,), jnp.int32),
          pltpu.VMEM((ROWS_PER_W, MEM_DIM), jnp.float32),
          pltpu.VMEM((B,), jnp.int32),
          pltpu.VMEM((B,), jnp.int32),
          pltpu.VMEM((NUM_NODES,), jnp.int32),
          pltpu.SemaphoreType.DMA,
      ],
  )
  def sc_gather(mem_hbm, ids_hbm, out_hbm, src_hbm, idx_v, rows_v, allids_v,
                src_v, w_v, sem):
    wid = _worker_id()
    base = wid * ROWS_PER_W
    pltpu.sync_copy(ids_hbm.at[pl.ds(base, ROWS_PER_W)], idx_v)
    pltpu.async_copy(mem_hbm.at[idx_v], rows_v, sem).wait()
    pltpu.sync_copy(rows_v, out_hbm.at[pl.ds(base, ROWS_PER_W)])

    @pl.when(wid == 0)
    def _dup():
      pltpu.sync_copy(ids_hbm, allids_v)
      lane = lax.iota(jnp.int32, L)

      def w_body(k, _):
        idvec = allids_v[pl.ds(k * L, L)]
        plsc.store_scatter(w_v, [idvec], lane + k * L)
        return 0

      lax.fori_loop(0, B // L, w_body, 0, unroll=8)

      def r_body(k, _):
        idvec = allids_v[pl.ds(k * L, L)]
        src_v[pl.ds(k * L, L)] = plsc.load_gather(w_v, [idvec])
        return 0

      lax.fori_loop(0, B // L, r_body, 0, unroll=8)
      pltpu.sync_copy(src_v, src_hbm)

  @functools.partial(
      pl.kernel,
      out_type=(),
      mesh=mesh,
      scratch_types=[
          pltpu.VMEM((ROWS_PER_W,), jnp.int32),
          pltpu.VMEM((ROWS_PER_W,), jnp.int32),
          pltpu.VMEM((ROWS_PER_W, MEM_DIM), jnp.float32),
          pltpu.SemaphoreType.DMA,
          pltpu.SemaphoreType.DMA,
      ],
  )
  def sc_scatter(normed_hbm, ids_hbm, src_hbm, table, idx_v, src_v, rows_v,
                 gsem, ssem):
    base = _worker_id() * ROWS_PER_W
    pltpu.sync_copy(ids_hbm.at[pl.ds(base, ROWS_PER_W)], idx_v)
    pltpu.sync_copy(src_hbm.at[pl.ds(base, ROWS_PER_W)], src_v)
    pltpu.async_copy(normed_hbm.at[src_v], rows_v, gsem).wait()
    pltpu.async_copy(rows_v, table.at[idx_v], ssem).wait()

  return sc_gather, sc_scatter


@functools.cache
def _get_tc_merged():
  mesh = pltpu.create_tensorcore_mesh("tc")

  @functools.partial(
      pl.kernel,
      out_type=jax.ShapeDtypeStruct((B, MEM_DIM), jnp.float32),
      mesh=mesh,
      scratch_types=[
          pltpu.VMEM((_D, _ROWS, MEM_DIM), jnp.float32),
          pltpu.VMEM((B, MSG_DIM), jnp.float32),
          pltpu.VMEM((B, MEM_DIM), jnp.float32),
          pltpu.VMEM((B, MEM_DIM), jnp.float32),
          pltpu.VMEM((C, PERIOD), jnp.float32),
          pltpu.VMEM((C, MEM_DIM), jnp.float32),
          pltpu.VMEM((1, MEM_DIM), jnp.float32),
          pltpu.VMEM((1, MEM_DIM), jnp.float32),
          pltpu.VMEM((1, MEM_DIM), jnp.float32),
          pltpu.SemaphoreType.DMA,
          pltpu.SemaphoreType.DMA,
          pltpu.SemaphoreType.DMA,
      ],
  )
  def tc_merged(mem, messages, gathered, cw, lw, lb, gamma, beta, table,
                normed_out, bufs, msg_v, gath_v, norm_v, cw_v, lw_v, lb_v,
                gamma_v, beta_v, rsem, wsem, ssem):
    stage = [
        pltpu.async_copy(messages, msg_v, ssem),
        pltpu.async_copy(gathered, gath_v, ssem),
        pltpu.async_copy(cw, cw_v, ssem),
        pltpu.async_copy(lw, lw_v, ssem),
        pltpu.async_copy(lb, lb_v, ssem),
        pltpu.async_copy(gamma, gamma_v, ssem),
        pltpu.async_copy(beta, beta_v, ssem),
    ]

    def rd(k):
      return pltpu.async_copy(
          mem.at[pl.ds(k * _ROWS, _ROWS)], bufs.at[lax.rem(k, _D)], rsem)

    def wr(k):
      return pltpu.async_copy(
          bufs.at[lax.rem(k, _D)], table.at[pl.ds(k * _ROWS, _ROWS)], wsem)

    for k in range(_D - 1):
      rd(k)
    for d in stage:
      d.wait()

    def compute(j):
      s = j * _BLK
      cwv = cw_v[...]
      v = 0.5 * (cwv[:, 1:2] + cwv[:, 2:3])
      w = v * lw_v[...]
      y = (
          jnp.dot(msg_v[pl.ds(s, _BLK), :], w[:MSG_DIM],
                  preferred_element_type=jnp.float32)
          + jnp.dot(gath_v[pl.ds(s, _BLK), :], w[MSG_DIM:],
                    preferred_element_type=jnp.float32)
          + lb_v[...]
      )
      mu = jnp.mean(y, axis=-1, keepdims=True)
      d = y - mu
      var = jnp.mean(d * d, axis=-1, keepdims=True)
      norm_v[pl.ds(s, _BLK), :] = (
          d * lax.rsqrt(var + 1e-5) * gamma_v[...] + beta_v[...])

    def rd_wait(k):
      pltpu.make_async_copy(
          mem.at[pl.ds(k * _ROWS, _ROWS)], bufs.at[lax.rem(k, _D)],
          rsem).wait()

    def wr_wait(k):
      pltpu.make_async_copy(
          bufs.at[lax.rem(k, _D)], table.at[pl.ds(k * _ROWS, _ROWS)],
          wsem).wait()

    def body2(k, _):
      rd_wait(k)
      wr(k)

      @pl.when(k < _NBLK)
      def _():
        compute(k)

      @pl.when(k >= 1)
      def _():
        wr_wait(k - 1)

      @pl.when(k + _D - 1 < _NSTEP)
      def _():
        rd(k + _D - 1)

      return 0

    lax.fori_loop(0, _NSTEP, body2, 0)
    wr_wait(_NSTEP - 1)
    pltpu.async_copy(norm_v, normed_out, ssem).wait()

  return tc_merged


def kernel(mem, messages, node_ids, conv_w, lin_w, lin_b, gamma, beta):
  _sc_gather, _sc_scatter = _get_sc_kernels()
  ids = node_ids.astype(jnp.int32)
  gathered, src = _sc_gather(mem, ids)
  table = jax.empty_ref(
      jax.ShapeDtypeStruct((NUM_NODES, MEM_DIM), jnp.float32))
  normed = _get_tc_merged()(
      mem, messages, gathered, conv_w.reshape(C, PERIOD), lin_w,
      lin_b.reshape(1, MEM_DIM), gamma.reshape(1, MEM_DIM),
      beta.reshape(1, MEM_DIM), table)
  _sc_scatter(normed, ids, src, table)
  return jax.freeze(table)


# speedup vs baseline: 1.3175x; 1.3175x over previous
import functools

import jax
import jax.numpy as jnp
from jax import lax
from jax.experimental import pallas as pl
from jax.experimental.pallas import tpu as pltpu
from jax.experimental.pallas import tpu_sc as plsc

NUM_NODES = 100000
MEM_DIM = 128
MSG_DIM = 100
B = 4096
PERIOD = 4
C = MSG_DIM + MEM_DIM

NC = 2
NS = 16
NW = NC * NS
ROWS_PER_W = B // NW
L = 16

_ROWS = 2500
_NSTEP = NUM_NODES // _ROWS
_D = 4
_BLK = 512
_NBLK = B // _BLK


def _worker_id():
  return lax.axis_index("s") * NC + lax.axis_index("c")


@functools.cache
def _get_sc_kernels():
  mesh = plsc.VectorSubcoreMesh(
      core_axis_name="c", subcore_axis_name="s", num_cores=NC)

  @functools.partial(
      pl.kernel,
      out_type=(
          jax.ShapeDtypeStruct((B, MEM_DIM), jnp.float32),
          jax.ShapeDtypeStruct((B,), jnp.int32),
      ),
      mesh=mesh,
      compiler_params=pltpu.CompilerParams(needs_layout_passes=False),
      scratch_types=[
          pltpu.VMEM((ROWS_PER_W,), jnp.int32),
          pltpu.VMEM((ROWS_PER_W, MEM_DIM), jnp.float32),
          pltpu.VMEM((B,), jnp.int32),
          pltpu.VMEM((B,), jnp.int32),
          pltpu.VMEM((NUM_NODES,), jnp.int32),
          pltpu.SemaphoreType.DMA,
      ],
  )
  def sc_gather(mem_hbm, ids_hbm, out_hbm, src_hbm, idx_v, rows_v, allids_v,
                src_v, w_v, sem):
    wid = _worker_id()
    base = wid * ROWS_PER_W
    pltpu.sync_copy(ids_hbm.at[pl.ds(base, ROWS_PER_W)], idx_v)
    pltpu.async_copy(mem_hbm.at[idx_v], rows_v, sem).wait()
    pltpu.sync_copy(rows_v, out_hbm.at[pl.ds(base, ROWS_PER_W)])

    @pl.when(wid == 0)
    def _dup():
      pltpu.sync_copy(ids_hbm, allids_v)
      lane = lax.iota(jnp.int32, L)

      def w_body(k, _):
        idvec = allids_v[pl.ds(k * L, L)]
        plsc.store_scatter(w_v, [idvec], lane + k * L)
        return 0

      lax.fori_loop(0, B // L, w_body, 0, unroll=8)

      def r_body(k, _):
        idvec = allids_v[pl.ds(k * L, L)]
        src_v[pl.ds(k * L, L)] = plsc.load_gather(w_v, [idvec])
        return 0

      lax.fori_loop(0, B // L, r_body, 0, unroll=8)
      pltpu.sync_copy(src_v, src_hbm)

  @functools.partial(
      pl.kernel,
      out_type=(),
      mesh=mesh,
      scratch_types=[
          pltpu.VMEM((ROWS_PER_W,), jnp.int32),
          pltpu.VMEM((ROWS_PER_W,), jnp.int32),
          pltpu.VMEM((ROWS_PER_W, MEM_DIM), jnp.float32),
          pltpu.SemaphoreType.DMA,
          pltpu.SemaphoreType.DMA,
      ],
  )
  def sc_scatter(normed_hbm, ids_hbm, src_hbm, table, idx_v, src_v, rows_v,
                 gsem, ssem):
    base = _worker_id() * ROWS_PER_W
    pltpu.sync_copy(ids_hbm.at[pl.ds(base, ROWS_PER_W)], idx_v)
    pltpu.sync_copy(src_hbm.at[pl.ds(base, ROWS_PER_W)], src_v)
    pltpu.async_copy(normed_hbm.at[src_v], rows_v, gsem).wait()
    pltpu.async_copy(rows_v, table.at[idx_v], ssem).wait()

  return sc_gather, sc_scatter


@functools.cache
def _get_tc_merged():
  mesh = pltpu.create_tensorcore_mesh("tc")

  @functools.partial(
      pl.kernel,
      out_type=jax.ShapeDtypeStruct((B, MEM_DIM), jnp.float32),
      mesh=mesh,
      scratch_types=[
          pltpu.VMEM((_D, _ROWS, MEM_DIM), jnp.float32),
          pltpu.VMEM((B, MSG_DIM), jnp.float32),
          pltpu.VMEM((B, MEM_DIM), jnp.float32),
          pltpu.VMEM((B, MEM_DIM), jnp.float32),
          pltpu.VMEM((C, PERIOD), jnp.float32),
          pltpu.VMEM((C, MEM_DIM), jnp.float32),
          pltpu.VMEM((1, MEM_DIM), jnp.float32),
          pltpu.VMEM((1, MEM_DIM), jnp.float32),
          pltpu.VMEM((1, MEM_DIM), jnp.float32),
          pltpu.SemaphoreType.DMA,
          pltpu.SemaphoreType.DMA,
          pltpu.SemaphoreType.DMA,
      ],
  )
  def tc_merged(mem, messages, gathered, cw, lw, lb, gamma, beta, table,
                normed_out, bufs, msg_v, gath_v, norm_v, cw_v, lw_v, lb_v,
                gamma_v, beta_v, rsem, wsem, ssem):
    stage = [
        pltpu.async_copy(messages, msg_v, ssem),
        pltpu.async_copy(gathered, gath_v, ssem),
        pltpu.async_copy(cw, cw_v, ssem),
        pltpu.async_copy(lw, lw_v, ssem),
        pltpu.async_copy(lb, lb_v, ssem),
        pltpu.async_copy(gamma, gamma_v, ssem),
        pltpu.async_copy(beta, beta_v, ssem),
    ]

    def rd(k):
      return pltpu.async_copy(
          mem.at[pl.ds(k * _ROWS, _ROWS)], bufs.at[k % _D], rsem)

    def wr(k):
      return pltpu.async_copy(
          bufs.at[k % _D], table.at[pl.ds(k * _ROWS, _ROWS)], wsem)

    for k in range(_D - 1):
      rd(k)
    for d in stage:
      d.wait()

    def compute(j):
      s = j * _BLK
      cwv = cw_v[...]
      v = 0.5 * (cwv[:, 1:2] + cwv[:, 2:3])
      w = v * lw_v[...]
      y = (
          jnp.dot(msg_v[pl.ds(s, _BLK), :], w[:MSG_DIM],
                  preferred_element_type=jnp.float32)
          + jnp.dot(gath_v[pl.ds(s, _BLK), :], w[MSG_DIM:],
                    preferred_element_type=jnp.float32)
          + lb_v[...]
      )
      mu = jnp.mean(y, axis=-1, keepdims=True)
      d = y - mu
      var = jnp.mean(d * d, axis=-1, keepdims=True)
      norm_v[pl.ds(s, _BLK), :] = (
          d * lax.rsqrt(var + 1e-5) * gamma_v[...] + beta_v[...])

    def rd_wait(k):
      pltpu.make_async_copy(
          mem.at[pl.ds(k * _ROWS, _ROWS)], bufs.at[k % _D], rsem).wait()

    def wr_wait(k):
      pltpu.make_async_copy(
          bufs.at[k % _D], table.at[pl.ds(k * _ROWS, _ROWS)], wsem).wait()

    for k in range(_NSTEP):
      rd_wait(k)
      wr(k)
      if k < _NBLK:
        compute(k)
      if k >= 1:
        wr_wait(k - 1)
      if k + _D - 1 < _NSTEP:
        rd(k + _D - 1)
    wr_wait(_NSTEP - 1)
    pltpu.async_copy(norm_v, normed_out, ssem).wait()

  return tc_merged


def kernel(mem, messages, node_ids, conv_w, lin_w, lin_b, gamma, beta):
  _sc_gather, _sc_scatter = _get_sc_kernels()
  ids = node_ids.astype(jnp.int32)
  gathered, src = _sc_gather(mem, ids)
  table = jax.empty_ref(
      jax.ShapeDtypeStruct((NUM_NODES, MEM_DIM), jnp.float32))
  normed = _get_tc_merged()(
      mem, messages, gathered, conv_w.reshape(C, PERIOD), lin_w,
      lin_b.reshape(1, MEM_DIM), gamma.reshape(1, MEM_DIM),
      beta.reshape(1, MEM_DIM), table)
  _sc_scatter(normed, ids, src, table)
  return jax.freeze(table)
